# Initial kernel scaffold; baseline (speedup 1.0000x reference)
import jax
import jax.numpy as jnp
from jax.experimental import pallas as pl


def kernel(input, target, impl_l, impl_r, dis_l, dis_r):
    raise NotImplementedError("write your pallas kernel here")


# speedup vs baseline: 1.8866x; 1.8866x over previous
import functools

import jax
import jax.numpy as jnp
from jax import lax
from jax.experimental import pallas as pl
from jax.experimental.pallas import tpu as pltpu
from jax.experimental.pallas import tpu_sc as plsc

_B = 2048
_C = 1500
_BLK_B = 256
_GRID = _B // _BLK_B

_NC = 2
_NS = 16
_NW = _NC * _NS
_IMPL_PER_W = 384
_DIS_PER_W = 128
_N_IMPL_REAL = 10000
_N_DIS_REAL = 4000


def _tc_body(x_ref, t_ref, base_ref, s_ref, h_ref):
    step = pl.program_id(0)
    x = x_ref[...]
    t = t_ref[...]
    p = jax.nn.sigmoid(x)
    bce = jnp.maximum(x, 0.0) - x * t + jnp.log1p(jnp.exp(-jnp.abs(x)))
    h_part = lax.dot_general(
        p, p, (((0,), (0,)), ((), ())),
        preferred_element_type=jnp.float32,
        precision=lax.Precision.HIGHEST,
    )

    @pl.when(step == 0)
    def _():
        base_ref[0, 0] = 0.0
        s_ref[...] = jnp.zeros_like(s_ref)
        h_ref[...] = jnp.zeros_like(h_ref)

    base_ref[0, 0] += jnp.sum(bce)
    s_ref[...] += jnp.sum(p, axis=0, keepdims=True)
    h_ref[...] += h_part


def _tc_stats(x, t):
    return pl.pallas_call(
        _tc_body,
        grid=(_GRID,),
        in_specs=[
            pl.BlockSpec((_BLK_B, _C), lambda i: (i, 0)),
            pl.BlockSpec((_BLK_B, _C), lambda i: (i, 0)),
        ],
        out_specs=[
            pl.BlockSpec((1, 1), lambda i: (0, 0), memory_space=pltpu.SMEM),
            pl.BlockSpec((1, _C), lambda i: (0, 0)),
            pl.BlockSpec((_C, _C), lambda i: (0, 0)),
        ],
        out_shape=[
            jax.ShapeDtypeStruct((1, 1), jnp.float32),
            jax.ShapeDtypeStruct((1, _C), jnp.float32),
            jax.ShapeDtypeStruct((_C, _C), jnp.float32),
        ],
    )(x, t)


@functools.cache
def _build_sc_pair_sums():
    mesh = plsc.VectorSubcoreMesh(
        core_axis_name="c", subcore_axis_name="s",
        num_cores=_NC, num_subcores=_NS,
    )
    return pl.kernel(
        _sc_pair_sums_body,
        out_type=jax.ShapeDtypeStruct((_NW, 3, 16), jnp.float32),
        mesh=mesh,
        scratch_types=[
            pltpu.VMEM((_IMPL_PER_W,), jnp.int32),
            pltpu.VMEM((_IMPL_PER_W,), jnp.int32),
            pltpu.VMEM((_DIS_PER_W,), jnp.int32),
            pltpu.VMEM((_DIS_PER_W,), jnp.int32),
            pltpu.VMEM((3, 128), jnp.int32),
            pltpu.VMEM((3, 128), jnp.int32),
            pltpu.VMEM((_DIS_PER_W,), jnp.int32),
            pltpu.VMEM((_IMPL_PER_W,), jnp.float32),
            pltpu.VMEM((_IMPL_PER_W,), jnp.float32),
            pltpu.VMEM((_DIS_PER_W,), jnp.float32),
            pltpu.VMEM((3, 16), jnp.float32),
            pltpu.SemaphoreType.DMA,
        ],
    )


def _sc_pair_sums_body(h_hbm, s_hbm, il_hbm, ir_hbm, dl_hbm, dr_hbm, out_hbm,
                       il_v, ir_v, dl_v, dr_v, fidx_v, sidx_v, didx_v,
                       hvals_i, svals_i, hvals_d, out_v, sem):
    wid = lax.axis_index("s") * _NC + lax.axis_index("c")
    ibase = wid * _IMPL_PER_W
    dbase = wid * _DIS_PER_W

    pltpu.sync_copy(il_hbm.at[pl.ds(ibase, _IMPL_PER_W)], il_v)
    pltpu.sync_copy(ir_hbm.at[pl.ds(ibase, _IMPL_PER_W)], ir_v)
    pltpu.sync_copy(dl_hbm.at[pl.ds(dbase, _DIS_PER_W)], dl_v)
    pltpu.sync_copy(dr_hbm.at[pl.ds(dbase, _DIS_PER_W)], dr_v)

    for i in range(_IMPL_PER_W // 16):
        l16 = il_v[pl.ds(i * 16, 16)]
        r16 = ir_v[pl.ds(i * 16, 16)]
        fidx_v[i // 8, pl.ds((i % 8) * 16, 16)] = l16 * _C + r16
        sidx_v[i // 8, pl.ds((i % 8) * 16, 16)] = l16
    for i in range(_DIS_PER_W // 16):
        l16 = dl_v[pl.ds(i * 16, 16)]
        r16 = dr_v[pl.ds(i * 16, 16)]
        didx_v[pl.ds(i * 16, 16)] = l16 * _C + r16

    copies = []
    for j in range(3):
        copies.append(pltpu.async_copy(
            h_hbm.at[fidx_v.at[j]], hvals_i.at[pl.ds(j * 128, 128)], sem))
        copies.append(pltpu.async_copy(
            s_hbm.at[sidx_v.at[j]], svals_i.at[pl.ds(j * 128, 128)], sem))
    copies.append(pltpu.async_copy(h_hbm.at[didx_v], hvals_d, sem))
    for c in copies:
        c.wait()

    acc_s = jnp.zeros((16,), jnp.float32)
    acc_ih = jnp.zeros((16,), jnp.float32)
    acc_dh = jnp.zeros((16,), jnp.float32)
    lanes = lax.iota(jnp.int32, 16)
    for i in range(_IMPL_PER_W // 16):
        valid = (ibase + i * 16 + lanes) < _N_IMPL_REAL
        acc_ih = acc_ih + jnp.where(valid, hvals_i[pl.ds(i * 16, 16)], 0.0)
        acc_s = acc_s + jnp.where(valid, svals_i[pl.ds(i * 16, 16)], 0.0)
    for i in range(_DIS_PER_W // 16):
        valid = (dbase + i * 16 + lanes) < _N_DIS_REAL
        acc_dh = acc_dh + jnp.where(valid, hvals_d[pl.ds(i * 16, 16)], 0.0)

    out_v[0, :] = acc_s
    out_v[1, :] = acc_ih
    out_v[2, :] = acc_dh
    pltpu.sync_copy(out_v, out_hbm.at[wid])


def kernel(input, target, impl_l, impl_r, dis_l, dis_r):
    base_sum, s2d, h = _tc_stats(input, target)

    h_flat = h.reshape(_C * _C)
    s1d = s2d.reshape(_C)
    il = jnp.zeros((_NW * _IMPL_PER_W,), jnp.int32).at[:_N_IMPL_REAL].set(impl_l)
    ir = jnp.zeros((_NW * _IMPL_PER_W,), jnp.int32).at[:_N_IMPL_REAL].set(impl_r)
    dl = jnp.zeros((_NW * _DIS_PER_W,), jnp.int32).at[:_N_DIS_REAL].set(dis_l)
    dr = jnp.zeros((_NW * _DIS_PER_W,), jnp.int32).at[:_N_DIS_REAL].set(dis_r)

    partials = _build_sc_pair_sums()(h_flat, s1d, il, ir, dl, dr)
    sums = jnp.sum(partials, axis=(0, 2))

    base_loss = base_sum[0, 0] / (_B * _C)
    implication_loss = (sums[0] - sums[1]) / _B
    disjointness_loss = sums[2] / _B
    loss = base_loss + 0.1 * implication_loss
    total = loss + 100.0 * disjointness_loss
    return (total, base_loss, implication_loss, disjointness_loss)
